# baseline (speedup 1.0000x reference)
import jax
import jax.numpy as jnp
from jax import lax
from jax.experimental import pallas as pl
from jax.experimental.pallas import tpu as pltpu
from jax.experimental.pallas import tpu_sc as plsc

_D = 64
_L = 16
_EPS = 1e-5
_NC = 2
_NS = 16
_NW = _NC * _NS
_CHUNK = 128

_GDN = lax.GatherDimensionNumbers(
    offset_dims=(), collapsed_slice_dims=(0,), start_index_map=(0,))


def _shuffle(v, p2d):
    return lax.gather(v, p2d, _GDN, slice_sizes=(1,),
                      mode=lax.GatherScatterMode.PROMISE_IN_BOUNDS)


def _ln_body(x_hbm, table_hbm, gamma_hbm, beta_hbm, out_hbm,
             idx_v, rows_v, gam_v, bet_v, sem):
    total = x_hbm.shape[0]
    rpw = total // _NW
    nchunk = rpw // _CHUNK
    wid = lax.axis_index("s") * _NC + lax.axis_index("c")
    base = wid * rpw

    pltpu.sync_copy(gamma_hbm, gam_v)
    pltpu.sync_copy(beta_hbm, bet_v)

    iota = lax.iota(jnp.int32, _L)
    perms = [(iota ^ jnp.int32(1 << k)).reshape(_L, 1) for k in range(4)]
    gs = [gam_v[pl.ds(d * _L, _L)] for d in range(4)]
    bs = [bet_v[pl.ds(d * _L, _L)] for d in range(4)]

    def chunk_body(g, carry):
        start = base + g * _CHUNK
        pltpu.sync_copy(x_hbm.at[pl.ds(start, _CHUNK)], idx_v)
        pltpu.async_copy(table_hbm.at[idx_v], rows_v, sem).wait()

        def row_body(r, c2):
            vs = [rows_v[r, pl.ds(d * _L, _L)] for d in range(4)]
            s = (vs[0] + vs[1]) + (vs[2] + vs[3])
            q = (vs[0] * vs[0] + vs[1] * vs[1]) + (vs[2] * vs[2] + vs[3] * vs[3])
            for p in perms:
                s = s + _shuffle(s, p)
                q = q + _shuffle(q, p)
            mean = s * jnp.float32(1.0 / _D)
            var = q * jnp.float32(1.0 / _D) - mean * mean + jnp.float32(_EPS)
            ibits = lax.bitcast_convert_type(var, jnp.int32)
            ibits = jnp.int32(0x5F3759DF) - lax.shift_right_arithmetic(
                ibits, jnp.full((_L,), 1, jnp.int32))
            y = lax.bitcast_convert_type(ibits, jnp.float32)
            half = var * jnp.float32(0.5)
            y = y * (jnp.float32(1.5) - half * y * y)
            y = y * (jnp.float32(1.5) - half * y * y)
            y = y * (jnp.float32(1.5) - half * y * y)
            for d in range(4):
                rows_v[r, pl.ds(d * _L, _L)] = (
                    (vs[d] - mean) * y * gs[d] + bs[d])
            return c2

        lax.fori_loop(0, _CHUNK, row_body, 0)
        pltpu.sync_copy(rows_v, out_hbm.at[pl.ds(start, _CHUNK)])
        return carry

    lax.fori_loop(0, nchunk, chunk_body, 0)


def kernel(x, table, gamma, beta):
    b, h = x.shape
    total = b * h
    xf = x.reshape(total)
    mesh = plsc.VectorSubcoreMesh(core_axis_name="c", subcore_axis_name="s")
    fn = pl.kernel(
        _ln_body,
        out_type=jax.ShapeDtypeStruct((total, _D), jnp.float32),
        mesh=mesh,
        compiler_params=pltpu.CompilerParams(use_tc_tiling_on_sc=False),
        scratch_types=[
            pltpu.VMEM((_CHUNK,), jnp.int32),
            pltpu.VMEM((_CHUNK, _D), jnp.float32),
            pltpu.VMEM((_D,), jnp.float32),
            pltpu.VMEM((_D,), jnp.float32),
            pltpu.SemaphoreType.DMA,
        ],
    )
    out = fn(xf, table, gamma, beta)
    return out.reshape(b, h, _D)


# speedup vs baseline: 1.7265x; 1.7265x over previous
import jax
import jax.numpy as jnp
from jax import lax
from jax.experimental import pallas as pl
from jax.experimental.pallas import tpu as pltpu
from jax.experimental.pallas import tpu_sc as plsc

_D = 64
_L = 16
_EPS = 1e-5
_NC = 2
_NS = 16
_NW = _NC * _NS
_CHUNK = 128
_NBUF = 8
_AHEAD = 4

_GDN = lax.GatherDimensionNumbers(
    offset_dims=(), collapsed_slice_dims=(0,), start_index_map=(0,))


def _shuffle(v, p2d):
    return lax.gather(v, p2d, _GDN, slice_sizes=(1,),
                      mode=lax.GatherScatterMode.PROMISE_IN_BOUNDS)


def _ln_body(x_hbm, table_hbm, gamma_hbm, beta_hbm, out_hbm,
             idx_all, rows, gsem, ssem, gam_v, bet_v):
    total = x_hbm.shape[0]
    rpw = total // _NW
    nchunk = rpw // _CHUNK
    wid = lax.axis_index("s") * _NC + lax.axis_index("c")
    base = wid * rpw

    pltpu.sync_copy(x_hbm.at[pl.ds(base, rpw)], idx_all)
    pltpu.sync_copy(gamma_hbm, gam_v)
    pltpu.sync_copy(beta_hbm, bet_v)

    iota = lax.iota(jnp.int32, _L)
    perms = [(iota ^ jnp.int32(1 << k)).reshape(_L, 1) for k in range(4)]
    gs = [gam_v[pl.ds(d * _L, _L)] for d in range(4)]
    bs = [bet_v[pl.ds(d * _L, _L)] for d in range(4)]

    def start_gather(g, b):
        pltpu.async_copy(
            table_hbm.at[idx_all.at[pl.ds(g * _CHUNK, _CHUNK)]],
            rows[b], gsem[b])

    def wait_gather(b):
        pltpu.make_async_copy(
            table_hbm.at[pl.ds(0, _CHUNK)], rows[b], gsem[b]).wait()

    def start_store(g, b):
        pltpu.async_copy(
            rows[b], out_hbm.at[pl.ds(base + g * _CHUNK, _CHUNK)], ssem[b])

    def wait_store(b):
        pltpu.make_async_copy(
            rows[b], out_hbm.at[pl.ds(0, _CHUNK)], ssem[b]).wait()

    def compute(b):
        rows_b = rows[b]

        @plsc.parallel_loop(0, _CHUNK, unroll=4)
        def _row(r):
            vs = [rows_b[r, pl.ds(d * _L, _L)] for d in range(4)]
            s = (vs[0] + vs[1]) + (vs[2] + vs[3])
            q = (vs[0] * vs[0] + vs[1] * vs[1]) + (
                vs[2] * vs[2] + vs[3] * vs[3])
            for p in perms:
                s = s + _shuffle(s, p)
                q = q + _shuffle(q, p)
            mean = s * jnp.float32(1.0 / _D)
            var = q * jnp.float32(1.0 / _D) - mean * mean + jnp.float32(_EPS)
            ibits = lax.bitcast_convert_type(var, jnp.int32)
            ibits = jnp.int32(0x5F3759DF) - lax.shift_right_arithmetic(
                ibits, jnp.full((_L,), 1, jnp.int32))
            y = lax.bitcast_convert_type(ibits, jnp.float32)
            half = var * jnp.float32(0.5)
            y = y * (jnp.float32(1.5) - half * y * y)
            y = y * (jnp.float32(1.5) - half * y * y)
            y = y * (jnp.float32(1.5) - half * y * y)
            for d in range(4):
                rows_b[r, pl.ds(d * _L, _L)] = (
                    (vs[d] - mean) * y * gs[d] + bs[d])

    for g in range(_AHEAD):
        start_gather(g, g)

    for db in range(_NBUF):
        wait_gather(db)
        compute(db)
        start_store(db, db)
        nb = (db + _AHEAD) % _NBUF
        if db >= _AHEAD:
            wait_store(nb)
        start_gather(db + _AHEAD, nb)

    @pl.loop(_NBUF, nchunk - _NBUF, step=_NBUF)
    def _blk(g0):
        for db in range(_NBUF):
            g = g0 + db
            wait_gather(db)
            compute(db)
            start_store(g, db)
            nb = (db + _AHEAD) % _NBUF
            wait_store(nb)
            start_gather(g + _AHEAD, nb)

    t0 = nchunk - _NBUF
    for db in range(_NBUF):
        g = t0 + db
        wait_gather(db)
        compute(db)
        start_store(g, db)
        if db < _AHEAD:
            nb = (db + _AHEAD) % _NBUF
            wait_store(nb)
            start_gather(g + _AHEAD, nb)

    for b in range(_NBUF):
        wait_store(b)


def kernel(x, table, gamma, beta):
    b, h = x.shape
    total = b * h
    xf = x.reshape(total)
    rpw = total // _NW
    mesh = plsc.VectorSubcoreMesh(core_axis_name="c", subcore_axis_name="s")
    fn = pl.kernel(
        _ln_body,
        out_type=jax.ShapeDtypeStruct((total, _D), jnp.float32),
        mesh=mesh,
        compiler_params=pltpu.CompilerParams(use_tc_tiling_on_sc=False),
        scratch_types=[
            pltpu.VMEM((rpw,), jnp.int32),
            [pltpu.VMEM((_CHUNK, _D), jnp.float32) for _ in range(_NBUF)],
            [pltpu.SemaphoreType.DMA for _ in range(_NBUF)],
            [pltpu.SemaphoreType.DMA for _ in range(_NBUF)],
            pltpu.VMEM((_D,), jnp.float32),
            pltpu.VMEM((_D,), jnp.float32),
        ],
    )
    out = fn(xf, table, gamma, beta)
    return out.reshape(b, h, _D)
